# baseline (speedup 1.0000x reference)
import functools

import jax
import jax.numpy as jnp
from jax import lax
from jax.experimental import pallas as pl
from jax.experimental.pallas import tpu as pltpu
from jax.experimental.pallas import tpu_sc as plsc


def _pair_hist_sc(s_idx, o_idx, num_obj):
    V, E = s_idx.shape
    Oh = num_obj // 2
    OP = 384
    NW = Oh * OP
    CH = E // 8
    mesh = plsc.VectorSubcoreMesh(core_axis_name="c", subcore_axis_name="s")

    @functools.partial(
        pl.kernel,
        mesh=mesh,
        out_type=jax.ShapeDtypeStruct((V, 2, NW), jnp.int32),
        scratch_types=[
            pltpu.VMEM((CH,), jnp.int32),
            pltpu.VMEM((CH,), jnp.int32),
            pltpu.VMEM((NW + 16,), jnp.int32),
            pltpu.VMEM((NW + 16,), jnp.int32),
        ],
    )
    def hist(s_hbm, o_hbm, out_hbm, s_v, o_v, acc_a, acc_b):
        wid = lax.axis_index("s") * 2 + lax.axis_index("c")

        z16 = jnp.zeros((16,), jnp.int32)

        def zero_body(i, carry):
            acc_a[pl.ds(i * 16, 16)] = z16
            acc_b[pl.ds(i * 16, 16)] = z16
            return carry

        lax.fori_loop(0, (NW + 16) // 16, zero_body, 0)

        iota16 = lax.iota(jnp.int32, 16)
        lane0 = jnp.where(iota16 == 0, 1, 0)

        def chunk_body(k, carry):
            pltpu.sync_copy(s_hbm.at[wid, pl.ds(k * CH, CH)], s_v)
            pltpu.sync_copy(o_hbm.at[wid, pl.ds(k * CH, CH)], o_v)

            def body(g, c2):
                sv = s_v[pl.ds(g * 16, 16)]
                ov = o_v[pl.ds(g * 16, 16)]
                hi = jnp.where(sv >= Oh, 1, 0)
                sm = sv - hi * Oh
                words = (ov >> 7) * (Oh * 128) + sm * 128 + (ov & 127)
                incs = jnp.where(hi == 1, 1 << 16, 1)
                for l in range(16):
                    w = words[l]
                    delta = lane0 * incs[l]
                    acc = acc_a if l % 2 == 0 else acc_b
                    acc[pl.ds(w, 16)] = acc[pl.ds(w, 16)] + delta
                return c2

            lax.fori_loop(0, CH // 16, body, 0)
            return carry

        lax.fori_loop(0, E // CH, chunk_body, 0)

        pltpu.sync_copy(acc_a.at[pl.ds(0, NW)], out_hbm.at[wid, 0])
        pltpu.sync_copy(acc_b.at[pl.ds(0, NW)], out_hbm.at[wid, 1])

    return hist(s_idx, o_idx)


def _merge_counts(c_blk):
    return c_blk[0] + c_blk[1]


def _cnt_src(w):
    wsum = jnp.sum(jnp.sum(w, axis=2, keepdims=True), axis=0)
    lo = (wsum & 0xFFFF).astype(jnp.float32)
    hi = (wsum >> 16).astype(jnp.float32)
    return jnp.concatenate([lo, hi], axis=0)


def _pad_rows(x, rows, dtype):
    return jnp.concatenate(
        [x.astype(dtype), jnp.zeros((rows, x.shape[1]), dtype)], axis=0)


def _stats1_body(x_ref, c_ref, stat_ref):
    v = pl.program_id(0)
    x = x_ref[0]
    w = _merge_counts(c_ref[0])
    o, d = x.shape
    nj = w.shape[0]
    x2 = x * x
    cnt_s = _cnt_src(w)
    s_src = jnp.sum(cnt_s * x, axis=0, keepdims=True)
    q_src = jnp.sum(cnt_s * x2, axis=0, keepdims=True)
    xp = _pad_rows(x, nj * 128 - o, jnp.float32)
    co = jnp.sum(w, axis=1, keepdims=True)
    cnt_o = ((co & 0xFFFF) + (co >> 16)).astype(jnp.float32)
    s_dst = jnp.zeros((1, d), jnp.float32)
    q_dst = jnp.zeros((1, d), jnp.float32)
    for j in range(nj):
        xj = xp[128 * j:128 * (j + 1)]
        s_dst = s_dst + jnp.dot(cnt_o[j], xj,
                                preferred_element_type=jnp.float32)
        q_dst = q_dst + jnp.dot(cnt_o[j], xj * xj,
                                preferred_element_type=jnp.float32)
    block = jnp.concatenate(
        [s_src, q_src, s_dst, q_dst, jnp.zeros((4, s_src.shape[1]), jnp.float32)],
        axis=0,
    )

    @pl.when(v == 0)
    def _():
        stat_ref[...] = block

    @pl.when(v > 0)
    def _():
        stat_ref[...] += block


def _pool_body(x_ref, c_ref, stat_ref, par_ref, w1_ref, pooled_ref, st2_ref, *, n1, d):
    v = pl.program_id(0)
    x = x_ref[0]
    w = _merge_counts(c_ref[0])
    o = x.shape[0]
    nj = w.shape[0]
    inv_n1 = 1.0 / n1
    mean_s = stat_ref[0:1] * inv_n1
    var_s = stat_ref[1:2] * inv_n1 - mean_s * mean_s
    mean_d = stat_ref[2:3] * inv_n1
    var_d = stat_ref[3:4] * inv_n1 - mean_d * mean_d
    scale_s = par_ref[0:1] * lax.rsqrt(var_s + 1e-5)
    scale_d = par_ref[1:2] * lax.rsqrt(var_d + 1e-5)
    shift_s = par_ref[2:3] - mean_s * scale_s
    shift_d = par_ref[3:4] - mean_d * scale_d
    c1 = par_ref[4:5]

    a_in = jnp.maximum(x * scale_s + shift_s, 0.0).astype(jnp.bfloat16)
    b_in = jnp.maximum(x * scale_d + shift_d, 0.0).astype(jnp.bfloat16)
    w1 = w1_ref[...].astype(jnp.bfloat16)
    A = jnp.dot(a_in, w1[0:d], preferred_element_type=jnp.float32)
    B = jnp.dot(b_in, w1[d:2 * d], preferred_element_type=jnp.float32)

    cnt = _cnt_src(w)
    lo_bf = (w & 0xFFFF).astype(jnp.float32).astype(jnp.bfloat16)
    hi_bf = (w >> 16).astype(jnp.float32).astype(jnp.bfloat16)
    Bp = _pad_rows(B, nj * 128 - o, jnp.bfloat16)
    oh = o // 2
    CB_lo = jnp.zeros((oh, d), jnp.float32)
    CB_hi = jnp.zeros((oh, d), jnp.float32)
    for j in range(nj):
        Bj = Bp[128 * j:128 * (j + 1)]
        CB_lo = CB_lo + jnp.dot(lo_bf[j], Bj,
                                preferred_element_type=jnp.float32)
        CB_hi = CB_hi + jnp.dot(hi_bf[j], Bj,
                                preferred_element_type=jnp.float32)
    CB = jnp.concatenate([CB_lo, CB_hi], axis=0)
    num = cnt * (A + c1) + CB
    pooled = num / jnp.maximum(cnt, 1.0)
    pooled_ref[0] = pooled

    s2 = jnp.sum(pooled, axis=0, keepdims=True)
    q2 = jnp.sum(pooled * pooled, axis=0, keepdims=True)
    block = jnp.concatenate(
        [s2, q2, jnp.zeros((6, s2.shape[1]), jnp.float32)], axis=0)

    @pl.when(v == 0)
    def _():
        st2_ref[...] = block

    @pl.when(v > 0)
    def _():
        st2_ref[...] += block


def _out_body(p_ref, st2_ref, par_ref, w2_ref, out_ref, *, n2):
    g, o, d = p_ref.shape
    p = p_ref[...].reshape(g * o, d)
    inv_n2 = 1.0 / n2
    mean2 = st2_ref[0:1] * inv_n2
    var2 = st2_ref[1:2] * inv_n2 - mean2 * mean2
    scale2 = par_ref[5:6] * lax.rsqrt(var2 + 1e-5)
    shift2 = par_ref[6:7] - mean2 * scale2
    y = jnp.maximum(p * scale2 + shift2, 0.0)
    out = jnp.dot(y, w2_ref[...], preferred_element_type=jnp.float32) + par_ref[7:8]
    out_ref[...] = out.reshape(g, o, d)


def _tc_pipeline(obj_vecs, C, params8, W1, W2, n1, interpret=False):
    V, O, D = obj_vecs.shape
    full = lambda shape: pl.BlockSpec(shape, lambda v: (0,) * len(shape))
    per_v = lambda shape: pl.BlockSpec((1,) + shape, lambda v: (v,) + (0,) * len(shape))

    stats1 = pl.pallas_call(
        _stats1_body,
        grid=(V,),
        in_specs=[per_v((O, D)), per_v((2, 3, O // 2, 128))],
        out_specs=full((8, D)),
        out_shape=jax.ShapeDtypeStruct((8, D), jnp.float32),
        interpret=interpret,
    )(obj_vecs, C)

    pooled, stats2 = pl.pallas_call(
        functools.partial(_pool_body, n1=float(n1), d=D),
        grid=(V,),
        in_specs=[per_v((O, D)), per_v((2, 3, O // 2, 128)), full((8, D)),
                  full((8, D)), full((2 * D, D))],
        out_specs=[per_v((O, D)), full((8, D))],
        out_shape=[jax.ShapeDtypeStruct((V, O, D), jnp.float32),
                   jax.ShapeDtypeStruct((8, D), jnp.float32)],
        interpret=interpret,
    )(obj_vecs, C, stats1, params8, W1)

    G3 = 4
    per_g = pl.BlockSpec((G3, O, D), lambda v: (v, 0, 0))
    out = pl.pallas_call(
        functools.partial(_out_body, n2=float(V * O)),
        grid=(V // G3,),
        in_specs=[per_g, full((8, D)), full((8, D)), full((D, D))],
        out_specs=per_g,
        out_shape=jax.ShapeDtypeStruct((V, O, D), jnp.float32),
        interpret=interpret,
    )(pooled, stats2, params8, W2)
    return out


def kernel(obj_vecs, edges, g1, b1, W1, c1, g2, b2, W2, c2):
    V, O, D = obj_vecs.shape
    E = edges.shape[1]
    s_idx = edges[:, :, 0]
    o_idx = edges[:, :, 2]
    params8 = jnp.stack([g1[:D], g1[D:], b1[:D], b1[D:], c1, g2, b2, c2], axis=0)

    C = _pair_hist_sc(s_idx, o_idx, O).reshape(V, 2, 3, O // 2, 128)

    return _tc_pipeline(obj_vecs, C, params8, W1, W2, V * E)


# speedup vs baseline: 1.0557x; 1.0557x over previous
import functools

import jax
import jax.numpy as jnp
from jax import lax
from jax.experimental import pallas as pl
from jax.experimental.pallas import tpu as pltpu
from jax.experimental.pallas import tpu_sc as plsc


def _pair_hist_sc(s_idx, o_idx, num_obj):
    V, E = s_idx.shape
    Oh = num_obj // 2
    NW = num_obj * Oh
    mesh = plsc.VectorSubcoreMesh(core_axis_name="c", subcore_axis_name="s")

    @functools.partial(
        pl.kernel,
        mesh=mesh,
        out_type=jax.ShapeDtypeStruct((V, 2, NW), jnp.int32),
        scratch_types=[
            pltpu.VMEM((E,), jnp.int32),
            pltpu.VMEM((E,), jnp.int32),
            pltpu.VMEM((NW + 16,), jnp.int32),
            pltpu.VMEM((NW + 16,), jnp.int32),
        ],
    )
    def hist(s_hbm, o_hbm, out_hbm, s_v, o_v, acc_a, acc_b):
        wid = lax.axis_index("s") * 2 + lax.axis_index("c")

        z16 = jnp.zeros((16,), jnp.int32)

        def zero_body(i, carry):
            acc_a[pl.ds(i * 16, 16)] = z16
            acc_b[pl.ds(i * 16, 16)] = z16
            return carry

        lax.fori_loop(0, (NW + 16) // 16, zero_body, 0)

        pltpu.sync_copy(s_hbm.at[wid], s_v)
        pltpu.sync_copy(o_hbm.at[wid], o_v)

        iota16 = lax.iota(jnp.int32, 16)
        lane0 = jnp.where(iota16 == 0, 1, 0)

        def body(g, carry):
            sv = s_v[pl.ds(g * 16, 16)]
            ov = o_v[pl.ds(g * 16, 16)]
            hi = jnp.where(ov >= Oh, 1, 0)
            words = sv * Oh + ov - hi * Oh
            incs = jnp.where(hi == 1, 1 << 16, 1)
            for l in range(16):
                w = words[l]
                delta = lane0 * incs[l]
                acc = acc_a if l % 2 == 0 else acc_b
                acc[pl.ds(w, 16)] = acc[pl.ds(w, 16)] + delta
            return carry

        lax.fori_loop(0, E // 16, body, 0)

        pltpu.sync_copy(acc_a.at[pl.ds(0, NW)], out_hbm.at[wid, 0])
        pltpu.sync_copy(acc_b.at[pl.ds(0, NW)], out_hbm.at[wid, 1])

    return hist(s_idx, o_idx)


def _unpack_counts(c_blk):
    w = c_blk[0] + c_blk[1]
    lo = (w & 0xFFFF).astype(jnp.float32)
    hi = (w >> 16).astype(jnp.float32)
    return lo, hi


def _stats1_body(x_ref, c_ref, stat_ref):
    v = pl.program_id(0)
    x = x_ref[0]
    lo, hi = _unpack_counts(c_ref[0])
    oh = lo.shape[1]
    x2 = x * x
    cnt_s = (jnp.sum(lo, axis=1, keepdims=True)
             + jnp.sum(hi, axis=1, keepdims=True))
    s_src = jnp.sum(cnt_s * x, axis=0, keepdims=True)
    q_src = jnp.sum(cnt_s * x2, axis=0, keepdims=True)
    co_lo = jnp.sum(lo, axis=0, keepdims=True)
    co_hi = jnp.sum(hi, axis=0, keepdims=True)
    s_dst = (jnp.dot(co_lo, x[:oh], preferred_element_type=jnp.float32)
             + jnp.dot(co_hi, x[oh:], preferred_element_type=jnp.float32))
    q_dst = (jnp.dot(co_lo, x2[:oh], preferred_element_type=jnp.float32)
             + jnp.dot(co_hi, x2[oh:], preferred_element_type=jnp.float32))
    block = jnp.concatenate(
        [s_src, q_src, s_dst, q_dst, jnp.zeros((4, s_src.shape[1]), jnp.float32)],
        axis=0,
    )

    @pl.when(v == 0)
    def _():
        stat_ref[...] = block

    @pl.when(v > 0)
    def _():
        stat_ref[...] += block


def _pool_body(x_ref, c_ref, stat_ref, par_ref, w1_ref, pooled_ref, st2_ref, *, n1, d):
    v = pl.program_id(0)
    x = x_ref[0]
    lo, hi = _unpack_counts(c_ref[0])
    oh = lo.shape[1]
    inv_n1 = 1.0 / n1
    mean_s = stat_ref[0:1] * inv_n1
    var_s = stat_ref[1:2] * inv_n1 - mean_s * mean_s
    mean_d = stat_ref[2:3] * inv_n1
    var_d = stat_ref[3:4] * inv_n1 - mean_d * mean_d
    scale_s = par_ref[0:1] * lax.rsqrt(var_s + 1e-5)
    scale_d = par_ref[1:2] * lax.rsqrt(var_d + 1e-5)
    shift_s = par_ref[2:3] - mean_s * scale_s
    shift_d = par_ref[3:4] - mean_d * scale_d
    c1 = par_ref[4:5]

    a_in = jnp.maximum(x * scale_s + shift_s, 0.0).astype(jnp.bfloat16)
    b_in = jnp.maximum(x * scale_d + shift_d, 0.0).astype(jnp.bfloat16)
    w1 = w1_ref[...].astype(jnp.bfloat16)
    A = jnp.dot(a_in, w1[0:d], preferred_element_type=jnp.float32)
    B = jnp.dot(b_in, w1[d:2 * d], preferred_element_type=jnp.float32)

    cnt = (jnp.sum(lo, axis=1, keepdims=True)
           + jnp.sum(hi, axis=1, keepdims=True))
    Bb = B.astype(jnp.bfloat16)
    CB = (jnp.dot(lo.astype(jnp.bfloat16), Bb[:oh],
                  preferred_element_type=jnp.float32)
          + jnp.dot(hi.astype(jnp.bfloat16), Bb[oh:],
                    preferred_element_type=jnp.float32))
    num = cnt * (A + c1) + CB
    pooled = num / jnp.maximum(cnt, 1.0)
    pooled_ref[0] = pooled

    s2 = jnp.sum(pooled, axis=0, keepdims=True)
    q2 = jnp.sum(pooled * pooled, axis=0, keepdims=True)
    block = jnp.concatenate(
        [s2, q2, jnp.zeros((6, s2.shape[1]), jnp.float32)], axis=0)

    @pl.when(v == 0)
    def _():
        st2_ref[...] = block

    @pl.when(v > 0)
    def _():
        st2_ref[...] += block


def _out_body(p_ref, st2_ref, par_ref, w2_ref, out_ref, *, n2):
    g, o, d = p_ref.shape
    p = p_ref[...].reshape(g * o, d)
    inv_n2 = 1.0 / n2
    mean2 = st2_ref[0:1] * inv_n2
    var2 = st2_ref[1:2] * inv_n2 - mean2 * mean2
    scale2 = par_ref[5:6] * lax.rsqrt(var2 + 1e-5)
    shift2 = par_ref[6:7] - mean2 * scale2
    y = jnp.maximum(p * scale2 + shift2, 0.0)
    out = jnp.dot(y, w2_ref[...], preferred_element_type=jnp.float32) + par_ref[7:8]
    out_ref[...] = out.reshape(g, o, d)


def _tc_pipeline(obj_vecs, C, params8, W1, W2, n1, interpret=False):
    V, O, D = obj_vecs.shape
    full = lambda shape: pl.BlockSpec(shape, lambda v: (0,) * len(shape))
    per_v = lambda shape: pl.BlockSpec((1,) + shape, lambda v: (v,) + (0,) * len(shape))

    stats1 = pl.pallas_call(
        _stats1_body,
        grid=(V,),
        in_specs=[per_v((O, D)), per_v((2, O, O // 2))],
        out_specs=full((8, D)),
        out_shape=jax.ShapeDtypeStruct((8, D), jnp.float32),
        interpret=interpret,
    )(obj_vecs, C)

    pooled, stats2 = pl.pallas_call(
        functools.partial(_pool_body, n1=float(n1), d=D),
        grid=(V,),
        in_specs=[per_v((O, D)), per_v((2, O, O // 2)), full((8, D)),
                  full((8, D)), full((2 * D, D))],
        out_specs=[per_v((O, D)), full((8, D))],
        out_shape=[jax.ShapeDtypeStruct((V, O, D), jnp.float32),
                   jax.ShapeDtypeStruct((8, D), jnp.float32)],
        interpret=interpret,
    )(obj_vecs, C, stats1, params8, W1)

    G3 = 4
    per_g = pl.BlockSpec((G3, O, D), lambda v: (v, 0, 0))
    out = pl.pallas_call(
        functools.partial(_out_body, n2=float(V * O)),
        grid=(V // G3,),
        in_specs=[per_g, full((8, D)), full((8, D)), full((D, D))],
        out_specs=per_g,
        out_shape=jax.ShapeDtypeStruct((V, O, D), jnp.float32),
        interpret=interpret,
    )(pooled, stats2, params8, W2)
    return out


def kernel(obj_vecs, edges, g1, b1, W1, c1, g2, b2, W2, c2):
    V, O, D = obj_vecs.shape
    E = edges.shape[1]
    s_idx = edges[:, :, 0]
    o_idx = edges[:, :, 2]
    params8 = jnp.stack([g1[:D], g1[D:], b1[:D], b1[D:], c1, g2, b2, c2], axis=0)

    C = _pair_hist_sc(s_idx, o_idx, O).reshape(V, 2, O, O // 2)

    return _tc_pipeline(obj_vecs, C, params8, W1, W2, V * E)
